# baseline (speedup 1.0000x reference)
import functools

import jax
import jax.numpy as jnp
from jax import lax
from jax.experimental import pallas as pl
from jax.experimental.pallas import tpu as pltpu
from jax.experimental.pallas import tpu_sc as plsc

_NUM_WORKERS = 32
_CHUNK = 128
_GROUP = 16


def _make_sc_kernel(B, D):
    rows_per_w = B // _NUM_WORKERS
    n_chunks = rows_per_w // _CHUNK
    n_groups = rows_per_w // _GROUP
    cols_len = D * rows_per_w

    mesh = plsc.VectorSubcoreMesh(core_axis_name="c", subcore_axis_name="s")

    @functools.partial(
        pl.kernel,
        out_type=jax.ShapeDtypeStruct((B,), jnp.float32),
        mesh=mesh,
        compiler_params=pltpu.CompilerParams(use_tc_tiling_on_sc=False),
        scratch_types=[
            pltpu.VMEM((n_chunks, _CHUNK), jnp.int32),
            pltpu.VMEM((n_chunks, _CHUNK), jnp.int32),
            pltpu.VMEM((cols_len,), jnp.float32),
            pltpu.VMEM((cols_len,), jnp.float32),
            pltpu.VMEM((rows_per_w,), jnp.float32),
            pltpu.VMEM((rows_per_w,), jnp.float32),
            pltpu.VMEM((16,), jnp.float32),
            pltpu.VMEM((rows_per_w,), jnp.float32),
            pltpu.SemaphoreType.DMA,
        ],
    )
    def svd_kernel(uidx_hbm, iidx_hbm, ut_hbm, it_hbm, ubias_hbm,
                   ibias_hbm, oab_hbm, out_hbm, uidx_v, iidx_v, ucols_v,
                   icols_v, ubias_v, ibias_v, oab_v, out_v, sem):
        wid = lax.axis_index("s") * 2 + lax.axis_index("c")

        pltpu.sync_copy(uidx_hbm.at[wid], uidx_v)
        pltpu.sync_copy(iidx_hbm.at[wid], iidx_v)
        pltpu.sync_copy(oab_hbm, oab_v.at[pl.ds(0, 1)])

        for j in range(n_chunks):
            rows = pl.ds(j * _CHUNK, _CHUNK)
            pltpu.async_copy(ubias_hbm.at[uidx_v.at[j]], ubias_v.at[rows], sem)
            pltpu.async_copy(ibias_hbm.at[iidx_v.at[j]], ibias_v.at[rows], sem)

        def fire(d, carry):
            for j in range(n_chunks):
                dst = pl.ds(d * rows_per_w + j * _CHUNK, _CHUNK)
                pltpu.async_copy(ut_hbm.at[d].at[uidx_v.at[j]],
                                 ucols_v.at[dst], sem)
                pltpu.async_copy(it_hbm.at[d].at[iidx_v.at[j]],
                                 icols_v.at[dst], sem)
            return carry

        lax.fori_loop(0, D, fire, 0)

        pltpu.make_async_copy(ubias_hbm.at[pl.ds(0, cols_len)], ucols_v, sem).wait()
        pltpu.make_async_copy(ibias_hbm.at[pl.ds(0, cols_len)], icols_v, sem).wait()
        pltpu.make_async_copy(ubias_hbm.at[pl.ds(0, rows_per_w)], ubias_v, sem).wait()
        pltpu.make_async_copy(ibias_hbm.at[pl.ds(0, rows_per_w)], ibias_v, sem).wait()

        oab = oab_v[pl.ds(0, 16)][0]

        def group_body(g, carry):
            base = g * _GROUP
            acc = ubias_v[pl.ds(base, 16)] + ibias_v[pl.ds(base, 16)] + oab
            for d in range(D):
                sl = pl.ds(d * rows_per_w + base, 16)
                acc += ucols_v[sl] * icols_v[sl]
            out_v[pl.ds(base, 16)] = acc
            return carry

        lax.fori_loop(0, n_groups, group_body, 0)

        pltpu.sync_copy(out_v, out_hbm.at[pl.ds(wid * rows_per_w, rows_per_w)])

    return svd_kernel


@jax.jit
def kernel(userIdx, itemIdx, uEmbd, iEmbd, uBias, iBias, overAllBias):
    B = userIdx.shape[0]
    D = uEmbd.shape[1]
    uidx = userIdx.astype(jnp.int32).reshape(_NUM_WORKERS, -1, _CHUNK)
    iidx = itemIdx.astype(jnp.int32).reshape(_NUM_WORKERS, -1, _CHUNK)
    sc = _make_sc_kernel(B, D)
    return sc(uidx, iidx, uEmbd.T, iEmbd.T, uBias.reshape(-1),
              iBias.reshape(-1), overAllBias.astype(jnp.float32))


# speedup vs baseline: 8.4737x; 8.4737x over previous
import functools

import jax
import jax.numpy as jnp
from jax import lax
from jax.experimental import pallas as pl
from jax.experimental.pallas import tpu as pltpu
from jax.experimental.pallas import tpu_sc as plsc

_NUM_WORKERS = 32
_CHUNK = 128
_GROUP = 16
_CP = pltpu.CompilerParams(use_tc_tiling_on_sc=False)


def _make_gather_kernel(B, D):
    rows_per_w = B // _NUM_WORKERS
    n_chunks = rows_per_w // _CHUNK
    mesh = plsc.VectorSubcoreMesh(core_axis_name="c", subcore_axis_name="s")

    @functools.partial(
        pl.kernel,
        out_type=jax.ShapeDtypeStruct((B, D), jnp.float32),
        mesh=mesh,
        compiler_params=_CP,
        scratch_types=[
            pltpu.VMEM((n_chunks, _CHUNK), jnp.int32),
            pltpu.VMEM((rows_per_w, D), jnp.float32),
            pltpu.SemaphoreType.DMA,
        ],
    )
    def gather_kernel(uidx_hbm, uembd_hbm, out_hbm, uidx_v, urows_v, sem):
        wid = lax.axis_index("s") * 2 + lax.axis_index("c")
        pltpu.sync_copy(uidx_hbm.at[wid], uidx_v)
        copies = []
        for j in range(n_chunks):
            rows = pl.ds(j * _CHUNK, _CHUNK)
            copies.append(pltpu.async_copy(
                uembd_hbm.at[uidx_v.at[j]], urows_v.at[rows], sem))
        for c in copies:
            c.wait()
        pltpu.sync_copy(urows_v, out_hbm.at[pl.ds(wid * rows_per_w, rows_per_w)])

    return gather_kernel


def _make_dot_kernel(B, D):
    rows_per_w = B // _NUM_WORKERS
    n_chunks = rows_per_w // _CHUNK
    n_groups = rows_per_w // _GROUP
    n_dim_chunks = D // 16
    mesh = plsc.VectorSubcoreMesh(core_axis_name="c", subcore_axis_name="s")

    @functools.partial(
        pl.kernel,
        out_type=jax.ShapeDtypeStruct((B,), jnp.float32),
        mesh=mesh,
        compiler_params=_CP,
        scratch_types=[
            pltpu.VMEM((n_chunks, _CHUNK), jnp.int32),
            pltpu.VMEM((n_chunks, _CHUNK), jnp.int32),
            pltpu.VMEM((rows_per_w, D), jnp.float32),
            pltpu.VMEM((rows_per_w, D), jnp.float32),
            pltpu.VMEM((rows_per_w,), jnp.float32),
            pltpu.VMEM((rows_per_w,), jnp.float32),
            pltpu.VMEM((16,), jnp.float32),
            pltpu.VMEM((3 * 512,), jnp.float32),
            pltpu.VMEM((rows_per_w,), jnp.float32),
            pltpu.SemaphoreType.DMA,
        ],
    )
    def dot_kernel(iidx_hbm, uidx_hbm, iembd_hbm, urows_hbm, ubias_hbm,
                   ibias_hbm, oab_hbm, out_hbm, iidx_v, uidx_v, urows_v,
                   irows_v, ubias_v, ibias_v, oab_v, fb_v, out_v, sem):
        wid = lax.axis_index("s") * 2 + lax.axis_index("c")
        base_row = wid * rows_per_w
        pltpu.sync_copy(iidx_hbm.at[wid], iidx_v)
        pltpu.sync_copy(uidx_hbm.at[wid], uidx_v)
        pltpu.sync_copy(oab_hbm, oab_v.at[pl.ds(0, 1)])

        copies = [pltpu.async_copy(
            urows_hbm.at[pl.ds(base_row, rows_per_w)], urows_v, sem)]
        for j in range(n_chunks):
            rows = pl.ds(j * _CHUNK, _CHUNK)
            copies.append(pltpu.async_copy(
                iembd_hbm.at[iidx_v.at[j]], irows_v.at[rows], sem))
            copies.append(pltpu.async_copy(
                ubias_hbm.at[uidx_v.at[j]], ubias_v.at[rows], sem))
            copies.append(pltpu.async_copy(
                ibias_hbm.at[iidx_v.at[j]], ibias_v.at[rows], sem))
        for c in copies:
            c.wait()

        iota16 = lax.iota(jnp.int32, 16)
        oab = oab_v[pl.ds(0, 16)][0]

        def group_body(g, carry):
            base = g * _GROUP
            res = ubias_v[pl.ds(base, 16)] + ibias_v[pl.ds(base, 16)] + oab
            dots = res * 0.0
            for r in range(_GROUP):
                row = base + r
                acc = urows_v[row, pl.ds(0, 16)] * irows_v[row, pl.ds(0, 16)]
                for cdim in range(1, n_dim_chunks):
                    sl = pl.ds(cdim * 16, 16)
                    acc += urows_v[row, sl] * irows_v[row, sl]
                f1 = acc + lax.rev(acc, (0,))
                fb_v[pl.ds(32 * r, 16)] = f1
                f2 = f1 + fb_v[pl.ds(32 * r + 4, 16)]
                fb_v[pl.ds(512 + 32 * r, 16)] = f2
                f3 = f2 + fb_v[pl.ds(512 + 32 * r + 2, 16)]
                fb_v[pl.ds(1024 + 32 * r, 16)] = f3
                f4 = f3 + fb_v[pl.ds(1024 + 32 * r + 1, 16)]
                dots = jnp.where(iota16 == r, f4[0], dots)
            out_v[pl.ds(base, 16)] = dots + res
            return carry

        lax.fori_loop(0, n_groups, group_body, 0)

        pltpu.sync_copy(out_v, out_hbm.at[pl.ds(base_row, rows_per_w)])

    return dot_kernel


@jax.jit
def kernel(userIdx, itemIdx, uEmbd, iEmbd, uBias, iBias, overAllBias):
    B = userIdx.shape[0]
    D = uEmbd.shape[1]
    uidx = userIdx.astype(jnp.int32).reshape(_NUM_WORKERS, -1, _CHUNK)
    iidx = itemIdx.astype(jnp.int32).reshape(_NUM_WORKERS, -1, _CHUNK)
    urows = _make_gather_kernel(B, D)(uidx, uEmbd)
    return _make_dot_kernel(B, D)(iidx, uidx, iEmbd, urows,
                                  uBias.reshape(-1), iBias.reshape(-1),
                                  overAllBias.astype(jnp.float32))


# speedup vs baseline: 8.7407x; 1.0315x over previous
import functools

import jax
import jax.numpy as jnp
from jax import lax
from jax.experimental import pallas as pl
from jax.experimental.pallas import tpu as pltpu
from jax.experimental.pallas import tpu_sc as plsc

_NUM_WORKERS = 32
_CHUNK = 128
_GROUP = 16
_CP = pltpu.CompilerParams(use_tc_tiling_on_sc=False)


def _make_gather_kernel(B, D):
    rows_per_w = B // _NUM_WORKERS
    n_chunks = rows_per_w // _CHUNK
    mesh = plsc.VectorSubcoreMesh(core_axis_name="c", subcore_axis_name="s")

    @functools.partial(
        pl.kernel,
        out_type=(jax.ShapeDtypeStruct((B, D), jnp.float32),
                  jax.ShapeDtypeStruct((B,), jnp.float32)),
        mesh=mesh,
        compiler_params=_CP,
        scratch_types=[
            pltpu.VMEM((n_chunks, _CHUNK), jnp.int32),
            pltpu.VMEM((rows_per_w, D), jnp.float32),
            pltpu.VMEM((rows_per_w,), jnp.float32),
            pltpu.SemaphoreType.DMA,
        ],
    )
    def gather_kernel(idx_hbm, embd_hbm, bias_hbm, rows_out_hbm, bias_out_hbm,
                      idx_v, rows_v, bias_v, sem):
        wid = lax.axis_index("s") * 2 + lax.axis_index("c")
        base_row = wid * rows_per_w
        pltpu.sync_copy(idx_hbm.at[wid], idx_v)
        copies = []
        for j in range(n_chunks):
            rows = pl.ds(j * _CHUNK, _CHUNK)
            copies.append(pltpu.async_copy(
                embd_hbm.at[idx_v.at[j]], rows_v.at[rows], sem))
            copies.append(pltpu.async_copy(
                bias_hbm.at[idx_v.at[j]], bias_v.at[rows], sem))
        for c in copies:
            c.wait()
        pltpu.sync_copy(rows_v, rows_out_hbm.at[pl.ds(base_row, rows_per_w)])
        pltpu.sync_copy(bias_v, bias_out_hbm.at[pl.ds(base_row, rows_per_w)])

    return gather_kernel


def _make_dot_kernel(B, D):
    rows_per_w = B // _NUM_WORKERS
    n_groups = rows_per_w // _GROUP
    n_dim_chunks = D // 16
    mesh = plsc.VectorSubcoreMesh(core_axis_name="c", subcore_axis_name="s")

    @functools.partial(
        pl.kernel,
        out_type=jax.ShapeDtypeStruct((B,), jnp.float32),
        mesh=mesh,
        compiler_params=_CP,
        scratch_types=[
            pltpu.VMEM((rows_per_w, D), jnp.float32),
            pltpu.VMEM((rows_per_w, D), jnp.float32),
            pltpu.VMEM((rows_per_w,), jnp.float32),
            pltpu.VMEM((rows_per_w,), jnp.float32),
            pltpu.VMEM((16,), jnp.float32),
            pltpu.VMEM((3 * 512,), jnp.float32),
            pltpu.VMEM((rows_per_w,), jnp.float32),
            pltpu.SemaphoreType.DMA,
        ],
    )
    def dot_kernel(urows_hbm, irows_hbm, ubias_hbm, ibias_hbm, oab_hbm,
                   out_hbm, urows_v, irows_v, ubias_v, ibias_v, oab_v,
                   fb_v, out_v, sem):
        wid = lax.axis_index("s") * 2 + lax.axis_index("c")
        base_row = wid * rows_per_w
        rows = pl.ds(base_row, rows_per_w)
        copies = [
            pltpu.async_copy(urows_hbm.at[rows], urows_v, sem),
            pltpu.async_copy(irows_hbm.at[rows], irows_v, sem),
            pltpu.async_copy(ubias_hbm.at[rows], ubias_v, sem),
            pltpu.async_copy(ibias_hbm.at[rows], ibias_v, sem),
        ]
        pltpu.sync_copy(oab_hbm, oab_v.at[pl.ds(0, 1)])
        for c in copies:
            c.wait()

        iota16 = lax.iota(jnp.int32, 16)
        oab = oab_v[pl.ds(0, 16)][0]

        def group_body(g, carry):
            base = g * _GROUP
            res = ubias_v[pl.ds(base, 16)] + ibias_v[pl.ds(base, 16)] + oab
            dots = res * 0.0
            for r in range(_GROUP):
                row = base + r
                acc = urows_v[row, pl.ds(0, 16)] * irows_v[row, pl.ds(0, 16)]
                for cdim in range(1, n_dim_chunks):
                    sl = pl.ds(cdim * 16, 16)
                    acc += urows_v[row, sl] * irows_v[row, sl]
                f1 = acc + lax.rev(acc, (0,))
                fb_v[pl.ds(32 * r, 16)] = f1
                f2 = f1 + fb_v[pl.ds(32 * r + 4, 16)]
                fb_v[pl.ds(512 + 32 * r, 16)] = f2
                f3 = f2 + fb_v[pl.ds(512 + 32 * r + 2, 16)]
                fb_v[pl.ds(1024 + 32 * r, 16)] = f3
                f4 = f3 + fb_v[pl.ds(1024 + 32 * r + 1, 16)]
                dots = jnp.where(iota16 == r, f4[0], dots)
            out_v[pl.ds(base, 16)] = dots + res
            return carry

        lax.fori_loop(0, n_groups, group_body, 0)

        pltpu.sync_copy(out_v, out_hbm.at[pl.ds(base_row, rows_per_w)])

    return dot_kernel


@jax.jit
def kernel(userIdx, itemIdx, uEmbd, iEmbd, uBias, iBias, overAllBias):
    B = userIdx.shape[0]
    D = uEmbd.shape[1]
    uidx = userIdx.astype(jnp.int32).reshape(_NUM_WORKERS, -1, _CHUNK)
    iidx = itemIdx.astype(jnp.int32).reshape(_NUM_WORKERS, -1, _CHUNK)
    gather = _make_gather_kernel(B, D)
    urows, ubg = gather(uidx, uEmbd, uBias.reshape(-1))
    irows, ibg = gather(iidx, iEmbd, iBias.reshape(-1))
    return _make_dot_kernel(B, D)(urows, irows, ubg, ibg,
                                  overAllBias.astype(jnp.float32))


# speedup vs baseline: 11.0099x; 1.2596x over previous
import functools

import jax
import jax.numpy as jnp
from jax import lax
from jax.experimental import pallas as pl
from jax.experimental.pallas import tpu as pltpu
from jax.experimental.pallas import tpu_sc as plsc

_NUM_WORKERS = 32
_CHUNK = 128
_GROUP = 16
_HALF = 256


def _make_bias_kernel(B):
    rows_per_w = B // _NUM_WORKERS
    n_chunks = rows_per_w // _CHUNK
    mesh = plsc.VectorSubcoreMesh(core_axis_name="c", subcore_axis_name="s")

    @functools.partial(
        pl.kernel,
        out_type=jax.ShapeDtypeStruct((B,), jnp.float32),
        mesh=mesh,
        compiler_params=pltpu.CompilerParams(use_tc_tiling_on_sc=False),
        scratch_types=[
            pltpu.VMEM((rows_per_w,), jnp.int32),
            pltpu.VMEM((rows_per_w,), jnp.int32),
            pltpu.VMEM((rows_per_w,), jnp.float32),
            pltpu.VMEM((rows_per_w,), jnp.float32),
            pltpu.VMEM((16,), jnp.float32),
            pltpu.VMEM((rows_per_w,), jnp.float32),
            pltpu.SemaphoreType.DMA,
        ],
    )
    def bias_kernel(uidx_hbm, iidx_hbm, ubt_hbm, ibt_hbm, oab_hbm, out_hbm,
                    uidx_v, iidx_v, ub_v, ib_v, oab_v, out_v, sem):
        wid = lax.axis_index("s") * 2 + lax.axis_index("c")
        base_row = wid * rows_per_w
        pltpu.sync_copy(uidx_hbm.at[pl.ds(base_row, rows_per_w)], uidx_v)
        pltpu.sync_copy(iidx_hbm.at[pl.ds(base_row, rows_per_w)], iidx_v)
        pltpu.sync_copy(oab_hbm, oab_v.at[pl.ds(0, 1)])
        copies = []
        for j in range(n_chunks):
            rows = pl.ds(j * _CHUNK, _CHUNK)
            copies.append(pltpu.async_copy(
                ubt_hbm.at[0].at[uidx_v.at[rows]], ub_v.at[rows], sem))
            copies.append(pltpu.async_copy(
                ibt_hbm.at[0].at[iidx_v.at[rows]], ib_v.at[rows], sem))
        for c in copies:
            c.wait()
        oab = oab_v[pl.ds(0, 16)][0]

        def body(g, carry):
            sl = pl.ds(g * _GROUP, 16)
            out_v[sl] = ub_v[sl] + ib_v[sl] + oab
            return carry

        lax.fori_loop(0, rows_per_w // _GROUP, body, 0)
        pltpu.sync_copy(out_v, out_hbm.at[pl.ds(base_row, rows_per_w)])

    return bias_kernel


def _make_dot_kernel(B, D):
    rows_per_w = B // _NUM_WORKERS
    n_halves = rows_per_w // _HALF
    n_groups = _HALF // _GROUP
    n_dim_chunks = D // 16
    mesh = plsc.VectorSubcoreMesh(core_axis_name="c", subcore_axis_name="s")

    @functools.partial(
        pl.kernel,
        out_type=jax.ShapeDtypeStruct((B,), jnp.float32),
        mesh=mesh,
        compiler_params=pltpu.CompilerParams(use_tc_tiling_on_sc=True),
        scratch_types=[
            pltpu.VMEM((rows_per_w,), jnp.int32),
            pltpu.VMEM((rows_per_w,), jnp.int32),
            pltpu.VMEM((_HALF, 2 * 64), jnp.float32),
            pltpu.VMEM((_HALF, 2 * 64), jnp.float32),
            pltpu.VMEM((rows_per_w,), jnp.float32),
            pltpu.VMEM((3 * 512,), jnp.float32),
            pltpu.VMEM((rows_per_w,), jnp.float32),
            pltpu.SemaphoreType.DMA,
        ],
    )
    def dot_kernel(uidx_hbm, iidx_hbm, big_hbm, bias_hbm, out_hbm,
                   uidx_v, iidx_v, urows_v, irows_v, bias_v, fb_v, out_v,
                   sem):
        wid = lax.axis_index("s") * 2 + lax.axis_index("c")
        base_row = wid * rows_per_w
        pltpu.sync_copy(uidx_hbm.at[pl.ds(base_row, rows_per_w)], uidx_v)
        pltpu.sync_copy(iidx_hbm.at[pl.ds(base_row, rows_per_w)], iidx_v)
        pltpu.sync_copy(bias_hbm.at[pl.ds(base_row, rows_per_w)], bias_v)

        iota16 = lax.iota(jnp.int32, 16)

        for h in range(n_halves):
            copies = []
            for j in range(_HALF // _CHUNK):
                src = pl.ds(h * _HALF + j * _CHUNK, _CHUNK)
                dst = pl.ds(j * _CHUNK, _CHUNK)
                copies.append(pltpu.async_copy(
                    big_hbm.at[uidx_v.at[src]], urows_v.at[dst], sem))
                copies.append(pltpu.async_copy(
                    big_hbm.at[iidx_v.at[src]], irows_v.at[dst], sem))
            for c in copies:
                c.wait()

            def group_body(g, carry):
                base = g * _GROUP
                res = bias_v[pl.ds(h * _HALF + base, 16)]
                dots = res * 0.0
                for r in range(_GROUP):
                    row = base + r
                    acc = (urows_v[row, pl.ds(0, 16)] *
                           irows_v[row, pl.ds(64, 16)])
                    for cdim in range(1, n_dim_chunks):
                        acc += (urows_v[row, pl.ds(cdim * 16, 16)] *
                                irows_v[row, pl.ds(64 + cdim * 16, 16)])
                    f1 = acc + lax.rev(acc, (0,))
                    fb_v[pl.ds(32 * r, 16)] = f1
                    f2 = f1 + fb_v[pl.ds(32 * r + 4, 16)]
                    fb_v[pl.ds(512 + 32 * r, 16)] = f2
                    f3 = f2 + fb_v[pl.ds(512 + 32 * r + 2, 16)]
                    fb_v[pl.ds(1024 + 32 * r, 16)] = f3
                    f4 = f3 + fb_v[pl.ds(1024 + 32 * r + 1, 16)]
                    dots = jnp.where(iota16 == r, f4[0], dots)
                out_v[pl.ds(h * _HALF + base, 16)] = dots + res
                return carry

            lax.fori_loop(0, n_groups, group_body, 0)

        pltpu.sync_copy(out_v, out_hbm.at[pl.ds(base_row, rows_per_w)])

    return dot_kernel


@jax.jit
def kernel(userIdx, itemIdx, uEmbd, iEmbd, uBias, iBias, overAllBias):
    B = userIdx.shape[0]
    D = uEmbd.shape[1]
    uidx = userIdx.astype(jnp.int32)
    iidx = itemIdx.astype(jnp.int32)
    big = jnp.concatenate([uEmbd, iEmbd], axis=1)
    bias_sums = _make_bias_kernel(B)(uidx, iidx, uBias.T, iBias.T,
                                     overAllBias.astype(jnp.float32))
    return _make_dot_kernel(B, D)(uidx, iidx, big, bias_sums)
